# baseline (speedup 1.0000x reference)
import jax
import jax.numpy as jnp
from jax import lax
from jax.experimental import pallas as pl
from jax.experimental.pallas import tpu as pltpu
from jax.experimental.pallas import tpu_sc as plsc

DIM = 32
SUB = 8
BATCH = 16384
NC, NS, L = 2, 16, 16
NW = NC * NS
BPW = BATCH // NW
CH = 32
NCH = BPW // CH


def _sc_body(aidx_hbm, bidx_hbm, a_hbm, b_hbm, out_hbm,
             aidx_v, bidx_v, abuf, bbuf, out_v, sema, semb):
    wid = lax.axis_index("s") * NC + lax.axis_index("c")
    base = wid * BPW

    pltpu.sync_copy(aidx_hbm.at[pl.ds(base, BPW)], aidx_v)
    pltpu.sync_copy(bidx_hbm.at[pl.ds(base, BPW)], bidx_v)

    iota = lax.iota(jnp.int32, L)

    def chunk(k, carry):
        coff = pl.multiple_of(k * CH, CH)
        copies = []
        raws = []
        for g in range(CH // L):
            sl = pl.ds(coff + g * L, L)
            raws.append((aidx_v[sl], bidx_v[sl]))
        for g, (araw, braw) in enumerate(raws):
            for j in range(L):
                i = g * L + j
                ba = pl.multiple_of(lax.bitwise_and(araw[j], -8), SUB)
                bb = pl.multiple_of(lax.bitwise_and(braw[j], -8), SUB)
                copies.append(
                    pltpu.async_copy(a_hbm.at[pl.ds(ba, SUB), :],
                                     abuf.at[i], sema))
                copies.append(
                    pltpu.async_copy(b_hbm.at[pl.ds(bb, SUB), :],
                                     bbuf.at[i], semb))
        for c in copies:
            c.wait()
        for g, (araw, braw) in enumerate(raws):
            acc = jnp.zeros((L,), jnp.float32)
            for j in range(L):
                i = g * L + j
                sa = lax.bitwise_and(araw[j], 7)
                sb = lax.bitwise_and(braw[j], 7)
                p = (abuf[i, sa, pl.ds(0, L)] * bbuf[i, sb, pl.ds(0, L)]
                     + abuf[i, sa, pl.ds(L, L)] * bbuf[i, sb, pl.ds(L, L)])
                acc = jnp.where(iota == j, jnp.sum(p), acc)
            out_v[pl.ds(coff + g * L, L)] = acc
        return carry

    lax.fori_loop(0, NCH, chunk, 0)

    pltpu.sync_copy(out_v, out_hbm.at[pl.ds(base, BPW)])


def kernel(aIdx, bIdx, A, B):
    num = A.shape[0]
    k = pl.kernel(
        _sc_body,
        out_type=jax.ShapeDtypeStruct((BATCH,), jnp.float32),
        mesh=plsc.VectorSubcoreMesh(core_axis_name="c", subcore_axis_name="s"),
        compiler_params=pltpu.CompilerParams(needs_layout_passes=False),
        scratch_types=[
            pltpu.VMEM((BPW,), jnp.int32),
            pltpu.VMEM((BPW,), jnp.int32),
            pltpu.VMEM((CH, SUB, DIM), jnp.float32),
            pltpu.VMEM((CH, SUB, DIM), jnp.float32),
            pltpu.VMEM((BPW,), jnp.float32),
            pltpu.SemaphoreType.DMA,
            pltpu.SemaphoreType.DMA,
        ],
    )
    del num
    return k(aIdx.astype(jnp.int32), bIdx.astype(jnp.int32), A, B)
